# baseline (device time: 182038 ns/iter reference)
import jax
import jax.numpy as jnp
from jax import lax
from jax.experimental import pallas as pl
from jax.experimental.pallas import tpu as pltpu

N_DEV = 16
N_SUB = 4


def kernel(x, w_mat):
    m_global, k_shard = x.shape
    _, n = w_mat.shape
    m_per = m_global // N_DEV
    ns = n // N_SUB

    def body(x_ref, w_ref, out_ref,
             comm0, comm1, comm2, comm3,
             acc0, acc1, acc2, acc3,
             ss0, rs0, ss1, rs1, ss2, rs2, ss3, rs3):
        my = lax.axis_index("i")
        left = lax.rem(my + N_DEV - 1, N_DEV)
        right = lax.rem(my + 1, N_DEV)

        chains = (
            (comm0, acc0, ss0, rs0, right, 0),
            (comm2, acc2, ss2, rs2, left, 2),
            (comm1, acc1, ss1, rs1, right, 1),
            (comm3, acc3, ss3, rs3, left, 3),
        )

        def partial(c, strip):
            return jnp.dot(
                x_ref[pl.ds(c * m_per, m_per), :],
                w_ref[:, strip * ns:(strip + 1) * ns],
                preferred_element_type=jnp.float32,
            )

        def rdma(chain, h):
            comm, _, ss, rs, dest, _ = chain
            return pltpu.make_async_remote_copy(
                src_ref=comm.at[h % 2],
                dst_ref=comm.at[(h + 1) % 2],
                send_sem=ss.at[h % 2],
                recv_sem=rs.at[(h + 1) % 2],
                device_id=(dest,),
                device_id_type=pl.DeviceIdType.MESH,
            )

        def recv_chunk(chain, h):
            if chain[4] is right:
                return lax.rem(my + 2 * N_DEV - 2 - h, N_DEV)
            return lax.rem(my + 2 + h, N_DEV)

        c_first_r = lax.rem(my + N_DEV - 1, N_DEV)
        c_first_l = lax.rem(my + 1, N_DEV)
        for chain in chains:
            comm, _, _, _, dest, strip = chain
            comm[0, :, :] = partial(
                c_first_r if dest is right else c_first_l, strip
            )

        barrier_sem = pltpu.get_barrier_semaphore()
        for nbr in (left, right):
            pl.semaphore_signal(
                barrier_sem, inc=1,
                device_id=(nbr,), device_id_type=pl.DeviceIdType.MESH,
            )
        pl.semaphore_wait(barrier_sem, 2)

        for chain in chains:
            rdma(chain, 0).start()
        for chain in chains:
            chain[1][:, :] = partial(recv_chunk(chain, 0), chain[5])

        for h in range(N_DEV - 1):
            last = h == N_DEV - 2
            for chain in chains:
                comm, acc, _, _, _, strip = chain
                rdma(chain, h).wait()
                if last:
                    out_ref[:, strip * ns:(strip + 1) * ns] = jnp.maximum(
                        comm[(h + 1) % 2, :, :] + acc[:, :], 0.0
                    )
                else:
                    comm[(h + 1) % 2, :, :] = comm[(h + 1) % 2, :, :] + acc[:, :]
                    rdma(chain, h + 1).start()
            if not last:
                for chain in chains:
                    chain[1][:, :] = partial(recv_chunk(chain, h + 1), chain[5])

    return pl.pallas_call(
        body,
        out_shape=jax.ShapeDtypeStruct((m_per, n), jnp.float32),
        in_specs=[
            pl.BlockSpec(memory_space=pltpu.VMEM),
            pl.BlockSpec(memory_space=pltpu.VMEM),
        ],
        out_specs=pl.BlockSpec(memory_space=pltpu.VMEM),
        scratch_shapes=[
            pltpu.VMEM((2, m_per, ns), jnp.float32),
            pltpu.VMEM((2, m_per, ns), jnp.float32),
            pltpu.VMEM((2, m_per, ns), jnp.float32),
            pltpu.VMEM((2, m_per, ns), jnp.float32),
            pltpu.VMEM((m_per, ns), jnp.float32),
            pltpu.VMEM((m_per, ns), jnp.float32),
            pltpu.VMEM((m_per, ns), jnp.float32),
            pltpu.VMEM((m_per, ns), jnp.float32),
            pltpu.SemaphoreType.DMA((2,)),
            pltpu.SemaphoreType.DMA((2,)),
            pltpu.SemaphoreType.DMA((2,)),
            pltpu.SemaphoreType.DMA((2,)),
            pltpu.SemaphoreType.DMA((2,)),
            pltpu.SemaphoreType.DMA((2,)),
            pltpu.SemaphoreType.DMA((2,)),
            pltpu.SemaphoreType.DMA((2,)),
        ],
        compiler_params=pltpu.CompilerParams(collective_id=0),
    )(x, w_mat)


# device time: 97696 ns/iter; 1.8633x vs baseline; 1.8633x over previous
import jax
import jax.numpy as jnp
from jax import lax
from jax.experimental import pallas as pl
from jax.experimental.pallas import tpu as pltpu

N_DEV = 16
N_SUB = 4


def kernel(x, w_mat):
    m_global, k_shard = x.shape
    _, n = w_mat.shape
    m_per = m_global // N_DEV
    ns = n // N_SUB

    def body(x_ref, w_ref, out_ref,
             comm0, comm1, comm2, comm3,
             acc0, acc1, acc2, acc3,
             ss0, rs0, ss1, rs1, ss2, rs2, ss3, rs3):
        my = lax.axis_index("i")
        left = lax.rem(my + N_DEV - 1, N_DEV)
        right = lax.rem(my + 1, N_DEV)

        chains = (
            (comm0, acc0, ss0, rs0, right, 0),
            (comm2, acc2, ss2, rs2, left, 2),
            (comm1, acc1, ss1, rs1, right, 1),
            (comm3, acc3, ss3, rs3, left, 3),
        )

        def partial(c, strip):
            return jnp.dot(
                x_ref[pl.ds(c * m_per, m_per), :],
                w_ref[:, strip * ns:(strip + 1) * ns],
                preferred_element_type=jnp.float32,
            )

        def rdma(chain, h):
            comm, _, ss, rs, dest, _ = chain
            return pltpu.make_async_remote_copy(
                src_ref=comm.at[h % 2],
                dst_ref=comm.at[(h + 1) % 2],
                send_sem=ss.at[h % 2],
                recv_sem=rs.at[(h + 1) % 2],
                device_id=(dest,),
                device_id_type=pl.DeviceIdType.MESH,
            )

        def recv_chunk(chain, h):
            if chain[4] is right:
                return lax.rem(my + 2 * N_DEV - 2 - h, N_DEV)
            return lax.rem(my + 2 + h, N_DEV)

        c_first_r = lax.rem(my + N_DEV - 1, N_DEV)
        c_first_l = lax.rem(my + 1, N_DEV)
        for chain in chains:
            comm, _, _, _, dest, strip = chain
            comm[0, :, :] = partial(
                c_first_r if dest is right else c_first_l, strip
            ).astype(jnp.bfloat16)

        barrier_sem = pltpu.get_barrier_semaphore()
        for nbr in (left, right):
            pl.semaphore_signal(
                barrier_sem, inc=1,
                device_id=(nbr,), device_id_type=pl.DeviceIdType.MESH,
            )
        pl.semaphore_wait(barrier_sem, 2)

        for chain in chains:
            rdma(chain, 0).start()
        for chain in chains:
            chain[1][:, :] = partial(recv_chunk(chain, 0), chain[5])

        for h in range(N_DEV - 1):
            last = h == N_DEV - 2
            for chain in chains:
                comm, acc, _, _, _, strip = chain
                rdma(chain, h).wait()
                if last:
                    out_ref[:, strip * ns:(strip + 1) * ns] = jnp.maximum(
                        comm[(h + 1) % 2, :, :].astype(jnp.float32) + acc[:, :],
                        0.0,
                    )
                else:
                    comm[(h + 1) % 2, :, :] = (
                        comm[(h + 1) % 2, :, :].astype(jnp.float32) + acc[:, :]
                    ).astype(jnp.bfloat16)
                    rdma(chain, h + 1).start()
            if not last:
                for chain in chains:
                    chain[1][:, :] = partial(recv_chunk(chain, h + 1), chain[5])

    return pl.pallas_call(
        body,
        out_shape=jax.ShapeDtypeStruct((m_per, n), jnp.float32),
        in_specs=[
            pl.BlockSpec(memory_space=pltpu.VMEM),
            pl.BlockSpec(memory_space=pltpu.VMEM),
        ],
        out_specs=pl.BlockSpec(memory_space=pltpu.VMEM),
        scratch_shapes=[
            pltpu.VMEM((2, m_per, ns), jnp.bfloat16),
            pltpu.VMEM((2, m_per, ns), jnp.bfloat16),
            pltpu.VMEM((2, m_per, ns), jnp.bfloat16),
            pltpu.VMEM((2, m_per, ns), jnp.bfloat16),
            pltpu.VMEM((m_per, ns), jnp.float32),
            pltpu.VMEM((m_per, ns), jnp.float32),
            pltpu.VMEM((m_per, ns), jnp.float32),
            pltpu.VMEM((m_per, ns), jnp.float32),
            pltpu.SemaphoreType.DMA((2,)),
            pltpu.SemaphoreType.DMA((2,)),
            pltpu.SemaphoreType.DMA((2,)),
            pltpu.SemaphoreType.DMA((2,)),
            pltpu.SemaphoreType.DMA((2,)),
            pltpu.SemaphoreType.DMA((2,)),
            pltpu.SemaphoreType.DMA((2,)),
            pltpu.SemaphoreType.DMA((2,)),
        ],
        compiler_params=pltpu.CompilerParams(collective_id=0),
    )(x, w_mat)
